# baseline (speedup 1.0000x reference)
import functools

import jax
import jax.numpy as jnp
from jax import lax
from jax.experimental import pallas as pl
from jax.experimental.pallas import tpu as pltpu
from jax.experimental.pallas import tpu_sc as plsc

NUM_REL = 20000
NUM_OBJ = 1000
NUM_REL_CLS = 51
NUM_OBJ_CLS = 151

NP = 20480
NT = 16
CH = NP // NT
LPT = CH // 16
NSUB = CH // 128
PROBW = 64


def _rowsum(e):
    rows, c = e.shape
    p = ((c + 7) // 8) * 8
    if p != c:
        e = jnp.concatenate([e, jnp.zeros((rows, p - c), e.dtype)], axis=1)
    acc = e[:, 0:8]
    for k in range(1, p // 8):
        acc = acc + e[:, 8 * k:8 * k + 8]
    s4 = acc[:, 0:4] + acc[:, 4:8]
    s2 = s4[:, 0:2] + s4[:, 2:4]
    return s2[:, 0:1] + s2[:, 1:2]


def _branch_body(x_ref, prob_ref, scores_ref, cls_ref):
    x = x_ref[...]
    m = jnp.max(x, axis=-1, keepdims=True)
    e = jnp.exp(x - m)
    p = e / _rowsum(e)
    pw = prob_ref.shape[-1]
    if pw != x.shape[1]:
        prob_ref[...] = jnp.concatenate(
            [p, jnp.zeros((x.shape[0], pw - x.shape[1]), p.dtype)], axis=1)
    else:
        prob_ref[...] = p
    cols = jax.lax.broadcasted_iota(jnp.int32, p.shape, 1)
    pm = jnp.where(cols >= 1, p, -1.0)
    sc = jnp.max(pm, axis=-1)
    scores_ref[...] = sc[:, None]
    cls_ref[...] = jnp.min(
        jnp.where(pm == sc[:, None], cols, x.shape[1]), axis=-1)[:, None]


_REL_BLK = 2000


def _dense_stage(rel_logits, obj_logits):
    rel_class_prob, rel_scores, rel_class = pl.pallas_call(
        _branch_body,
        grid=(NUM_REL // _REL_BLK,),
        in_specs=[pl.BlockSpec((_REL_BLK, NUM_REL_CLS), lambda i: (i, 0))],
        out_specs=(
            pl.BlockSpec((_REL_BLK, PROBW), lambda i: (i, 0)),
            pl.BlockSpec((_REL_BLK, 1), lambda i: (i, 0)),
            pl.BlockSpec((_REL_BLK, 1), lambda i: (i, 0)),
        ),
        out_shape=(
            jax.ShapeDtypeStruct((NUM_REL, PROBW), jnp.float32),
            jax.ShapeDtypeStruct((NUM_REL, 1), jnp.float32),
            jax.ShapeDtypeStruct((NUM_REL, 1), jnp.int32),
        ),
    )(rel_logits)
    rel_scores = rel_scores[:, 0]
    rel_class = rel_class[:, 0]
    _, obj_scores, obj_pred = pl.pallas_call(
        _branch_body,
        out_shape=(
            jax.ShapeDtypeStruct((NUM_OBJ, NUM_OBJ_CLS), jnp.float32),
            jax.ShapeDtypeStruct((NUM_OBJ, 1), jnp.float32),
            jax.ShapeDtypeStruct((NUM_OBJ, 1), jnp.int32),
        ),
    )(obj_logits)
    obj_scores = obj_scores[:, 0]
    obj_pred = obj_pred[:, 0]
    return rel_class_prob, rel_scores, rel_class, obj_scores, obj_pred


VALID_LAST = NUM_REL - (NT - 1) * CH


def _sc_sort_gather(rel_scores, obj_scores, pair_flat, rel_class, prob):
    mesh = plsc.VectorSubcoreMesh(
        core_axis_name="c", subcore_axis_name="s", num_cores=1)

    @functools.partial(
        pl.kernel,
        out_type=(
            jax.ShapeDtypeStruct((NP,), jnp.float32),
            jax.ShapeDtypeStruct((2 * NP,), jnp.int32),
            jax.ShapeDtypeStruct((NP,), jnp.int32),
            jax.ShapeDtypeStruct((NP, PROBW), jnp.float32),
        ),
        mesh=mesh,
        compiler_params=pltpu.CompilerParams(
            needs_layout_passes=False, use_tc_tiling_on_sc=False),
        scratch_types=[
            pltpu.VMEM((NUM_OBJ,), jnp.float32),
            pltpu.VMEM((CH,), jnp.float32),
            pltpu.VMEM((2 * CH,), jnp.int32),
            pltpu.VMEM((2 * NSUB, 128), jnp.int32),
            pltpu.VMEM((CH,), jnp.int32),
            pltpu.VMEM((CH,), jnp.int32),
            pltpu.VMEM((4096,), jnp.int32),
            pltpu.VMEM((256,), jnp.int32),
            pltpu.VMEM((NT, 256), jnp.int32),
            pltpu.VMEM((4096,), jnp.int32),
            pltpu.VMEM((4096,), jnp.int32),
            pltpu.VMEM((NSUB, 128), jnp.int32),
            pltpu.VMEM((NSUB, 128), jnp.int32),
            pltpu.VMEM((CH,), jnp.float32),
            pltpu.VMEM((CH,), jnp.int32),
            pltpu.VMEM((CH, PROBW), jnp.float32),
            pltpu.VMEM_SHARED((NP,), jnp.int32),
            pltpu.VMEM_SHARED((NP,), jnp.int32),
            pltpu.VMEM_SHARED((NP,), jnp.int32),
            pltpu.VMEM_SHARED((NP,), jnp.int32),
            pltpu.VMEM_SHARED((NT, 256), jnp.int32),
        ],
    )
    def k(rs_hbm, obj_hbm, pair_hbm, cls_hbm, prob_hbm,
          t_out, pair_out, cls_out, prob_out,
          obj_v, rs_v, pair2_v, pidx2_v, key_v, val_v,
          hist_v, tt_v, grid_v, scanrow_v, offs_v,
          pos2_v, idx2_v, f_v, g_v, rows_v,
          ka_sh, kb_sh, va_sh, vb_sh, hist_sh):
        t = lax.axis_index("s")
        base = t * CH
        lane = lax.iota(jnp.int32, 16)
        ones16 = jnp.ones((16,), jnp.int32)
        zeros16 = jnp.zeros((16,), jnp.int32)

        pltpu.sync_copy(obj_hbm, obj_v)

        @pl.when(t < NT - 1)
        def _full_load():
            pltpu.sync_copy(rs_hbm.at[pl.ds(base, CH)], rs_v)
            pltpu.sync_copy(pair_hbm.at[pl.ds(2 * base, 2 * CH)], pair2_v)

        @pl.when(t == NT - 1)
        def _tail_load():
            pltpu.sync_copy(rs_hbm.at[pl.ds(base, VALID_LAST)],
                            rs_v.at[pl.ds(0, VALID_LAST)])
            pltpu.sync_copy(pair_hbm.at[pl.ds(2 * base, 2 * VALID_LAST)],
                            pair2_v.at[pl.ds(0, 2 * VALID_LAST)])

        def ph0(j, carry):
            epos = j * 16 + lane
            gidx = base + epos
            pad = gidx >= NUM_REL
            i0 = plsc.load_gather(pair2_v, [2 * epos])
            i1 = plsc.load_gather(pair2_v, [2 * epos + 1])
            i0 = jnp.where(pad, zeros16, i0)
            i1 = jnp.where(pad, zeros16, i1)
            s0 = plsc.load_gather(obj_v, [i0])
            s1 = plsc.load_gather(obj_v, [i1])
            rs = plsc.load_gather(rs_v, [epos])
            tr = (rs * s0) * s1
            bits = plsc.bitcast(tr, jnp.int32)
            key = jnp.where(pad, jnp.int32(0x7FFFFFFF),
                            jnp.int32(0x7FFFFFFF) - bits)
            plsc.store_scatter(key_v, [epos], key)
            plsc.store_scatter(val_v, [epos], jnp.where(pad, zeros16, gidx))
            return carry
        lax.fori_loop(0, LPT * 16 // 16, ph0, jnp.int32(0))

        bufs = [(ka_sh, va_sh), (kb_sh, vb_sh)]
        for p in range(4):
            dst = bufs[p % 2]
            if p > 0:
                src = bufs[(p - 1) % 2]
                pltpu.sync_copy(src[0].at[pl.ds(base, CH)], key_v)
                pltpu.sync_copy(src[1].at[pl.ds(base, CH)], val_v)
            shift = 8 * p

            def clr(i, carry):
                plsc.store_scatter(hist_v, [i * 16 + lane], zeros16)
                return carry
            lax.fori_loop(0, 256, clr, jnp.int32(0))

            def hst(j, carry, shift=shift):
                epos = lane * LPT + j
                kk = plsc.load_gather(key_v, [epos])
                d = jnp.bitwise_and(lax.shift_right_logical(kk, shift), 255)
                plsc.addupdate_scatter(hist_v, [d * 16 + lane], ones16)
                return carry
            lax.fori_loop(0, LPT, hst, jnp.int32(0))

            def tot(g, carry):
                dvec = g * 16 + lane
                acc = zeros16
                for l in range(16):
                    acc = acc + plsc.load_gather(hist_v, [dvec * 16 + l])
                plsc.store_scatter(tt_v, [dvec], acc)
                return carry
            lax.fori_loop(0, 16, tot, jnp.int32(0))

            pltpu.sync_copy(tt_v, hist_sh.at[t])
            plsc.subcore_barrier()
            pltpu.sync_copy(hist_sh, grid_v)

            def scn(d, running):
                v = plsc.load_gather(grid_v, [lane, zeros16 + d])
                cs = plsc.cumsum(v)
                basev = (cs - v) + running
                plsc.store_scatter(scanrow_v, [d * 16 + lane], basev)
                lane_h = plsc.load_gather(hist_v, [d * 16 + lane])
                lane_cs = plsc.cumsum(lane_h)
                own = plsc.load_gather(scanrow_v, [zeros16 + (d * 16 + t)])
                plsc.store_scatter(offs_v, [d * 16 + lane],
                                   own + (lane_cs - lane_h))
                return running + jnp.sum(v)
            lax.fori_loop(0, 256, scn, jnp.int32(0))

            def prm(j, carry, shift=shift):
                epos = lane * LPT + j
                kk = plsc.load_gather(key_v, [epos])
                d = jnp.bitwise_and(lax.shift_right_logical(kk, shift), 255)
                oidx = d * 16 + lane
                pos = plsc.load_gather(offs_v, [oidx])
                plsc.store_scatter(offs_v, [oidx], pos + 1)
                plsc.store_scatter(
                    pos2_v,
                    [lax.shift_right_logical(epos, 7),
                     jnp.bitwise_and(epos, 127)], pos)
                return carry
            lax.fori_loop(0, LPT, prm, jnp.int32(0))

            for i in range(NSUB):
                pltpu.sync_copy(key_v.at[pl.ds(i * 128, 128)],
                                dst[0].at[pos2_v.at[i]])
                pltpu.sync_copy(val_v.at[pl.ds(i * 128, 128)],
                                dst[1].at[pos2_v.at[i]])
            plsc.subcore_barrier()

        fk, fv = bufs[3 % 2]
        pltpu.sync_copy(fk.at[pl.ds(base, CH)], key_v)
        pltpu.sync_copy(fv.at[pl.ds(base, CH)], val_v)

        def fin(j, carry):
            epos = j * 16 + lane
            kk = plsc.load_gather(key_v, [epos])
            tr = plsc.bitcast(jnp.int32(0x7FFFFFFF) - kk, jnp.float32)
            plsc.store_scatter(f_v, [epos], tr)
            vv = plsc.load_gather(val_v, [epos])
            plsc.store_scatter(
                idx2_v,
                [lax.shift_right_logical(epos, 7),
                 jnp.bitwise_and(epos, 127)], vv)
            e2 = 2 * epos
            plsc.store_scatter(
                pidx2_v,
                [lax.shift_right_logical(e2, 7), jnp.bitwise_and(e2, 127)],
                2 * vv)
            e2b = e2 + 1
            plsc.store_scatter(
                pidx2_v,
                [lax.shift_right_logical(e2b, 7), jnp.bitwise_and(e2b, 127)],
                2 * vv + 1)
            return carry
        lax.fori_loop(0, CH // 16, fin, jnp.int32(0))
        pltpu.sync_copy(f_v, t_out.at[pl.ds(base, CH)])

        for i in range(NSUB):
            pltpu.sync_copy(cls_hbm.at[idx2_v.at[i]],
                            g_v.at[pl.ds(i * 128, 128)])
        pltpu.sync_copy(g_v, cls_out.at[pl.ds(base, CH)])
        for i in range(2 * NSUB):
            pltpu.sync_copy(pair_hbm.at[pidx2_v.at[i]],
                            pair2_v.at[pl.ds(i * 128, 128)])
        pltpu.sync_copy(pair2_v, pair_out.at[pl.ds(2 * base, 2 * CH)])
        for i in range(NSUB):
            pltpu.sync_copy(prob_hbm.at[idx2_v.at[i]],
                            rows_v.at[pl.ds(i * 128, 128)])
        pltpu.sync_copy(rows_v, prob_out.at[pl.ds(base, CH)])

    return k(rel_scores, obj_scores, pair_flat, rel_class, prob)


def kernel(rel_logits, obj_logits, rel_pair_idxs):
    rel_class_prob, rel_scores, rel_class, obj_scores, obj_pred = _dense_stage(
        rel_logits, obj_logits)
    ts, pairs, clss, probs = _sc_sort_gather(
        rel_scores, obj_scores, rel_pair_idxs.reshape(-1), rel_class,
        rel_class_prob)
    pair_sorted = pairs.reshape(NP, 2)[:NUM_REL]
    return (obj_pred, obj_scores, pair_sorted,
            probs[:NUM_REL, :NUM_REL_CLS], clss[:NUM_REL], ts[:NUM_REL])


# speedup vs baseline: 1.1922x; 1.1922x over previous
import functools

import jax
import jax.numpy as jnp
from jax import lax
from jax.experimental import pallas as pl
from jax.experimental.pallas import tpu as pltpu
from jax.experimental.pallas import tpu_sc as plsc

NUM_REL = 20000
NUM_OBJ = 1000
NUM_REL_CLS = 51
NUM_OBJ_CLS = 151

NP = 20480
NT = 16
CH = NP // NT
LPT = CH // 16
NSUB = CH // 128
PROBW = 64
VALID_LAST = NUM_REL - (NT - 1) * CH

_SC_PARAMS = pltpu.CompilerParams(
    needs_layout_passes=False, use_tc_tiling_on_sc=False)


def _rowsum(e):
    rows, c = e.shape
    p = ((c + 7) // 8) * 8
    if p != c:
        e = jnp.concatenate([e, jnp.zeros((rows, p - c), e.dtype)], axis=1)
    acc = e[:, 0:8]
    for k in range(1, p // 8):
        acc = acc + e[:, 8 * k:8 * k + 8]
    s4 = acc[:, 0:4] + acc[:, 4:8]
    s2 = s4[:, 0:2] + s4[:, 2:4]
    return s2[:, 0:1] + s2[:, 1:2]


def _scores_body(x_ref, scores_ref, cls_ref):
    x = x_ref[...]
    m = jnp.max(x, axis=-1, keepdims=True)
    e = jnp.exp(x - m)
    p = e / _rowsum(e)
    cols = jax.lax.broadcasted_iota(jnp.int32, p.shape, 1)
    pm = jnp.where(cols >= 1, p, -1.0)
    sc = jnp.max(pm, axis=-1)
    scores_ref[...] = sc[:, None]
    cls_ref[...] = jnp.min(
        jnp.where(pm == sc[:, None], cols, x.shape[1]), axis=-1)[:, None]


def _prob_body(x_ref, prob_ref):
    x = x_ref[...]
    m = jnp.max(x, axis=-1, keepdims=True)
    e = jnp.exp(x - m)
    p = e / _rowsum(e)
    prob_ref[...] = jnp.concatenate(
        [p, jnp.zeros((x.shape[0], PROBW - x.shape[1]), p.dtype)], axis=1)


_REL_BLK = 2000


def _scores_stage(rel_logits, obj_logits):
    rel_scores, rel_class = pl.pallas_call(
        _scores_body,
        grid=(NUM_REL // _REL_BLK,),
        in_specs=[pl.BlockSpec((_REL_BLK, NUM_REL_CLS), lambda i: (i, 0))],
        out_specs=(
            pl.BlockSpec((_REL_BLK, 1), lambda i: (i, 0)),
            pl.BlockSpec((_REL_BLK, 1), lambda i: (i, 0)),
        ),
        out_shape=(
            jax.ShapeDtypeStruct((NUM_REL, 1), jnp.float32),
            jax.ShapeDtypeStruct((NUM_REL, 1), jnp.int32),
        ),
    )(rel_logits)
    obj_scores, obj_pred = pl.pallas_call(
        _scores_body,
        out_shape=(
            jax.ShapeDtypeStruct((NUM_OBJ, 1), jnp.float32),
            jax.ShapeDtypeStruct((NUM_OBJ, 1), jnp.int32),
        ),
    )(obj_logits)
    return (rel_scores[:, 0], rel_class[:, 0], obj_scores[:, 0],
            obj_pred[:, 0])


def _prob_stage(rel_logits):
    return pl.pallas_call(
        _prob_body,
        grid=(NUM_REL // _REL_BLK,),
        in_specs=[pl.BlockSpec((_REL_BLK, NUM_REL_CLS), lambda i: (i, 0))],
        out_specs=pl.BlockSpec((_REL_BLK, PROBW), lambda i: (i, 0)),
        out_shape=jax.ShapeDtypeStruct((NUM_REL, PROBW), jnp.float32),
    )(rel_logits)


def _sc_sort(rel_scores, obj_scores, p0, p1):
    mesh = plsc.VectorSubcoreMesh(
        core_axis_name="c", subcore_axis_name="s", num_cores=1)

    @functools.partial(
        pl.kernel,
        out_type=(
            jax.ShapeDtypeStruct((NP,), jnp.float32),
            jax.ShapeDtypeStruct((NP,), jnp.int32),
        ),
        mesh=mesh,
        compiler_params=_SC_PARAMS,
        scratch_types=[
            pltpu.VMEM((NUM_OBJ,), jnp.float32),
            pltpu.VMEM((CH,), jnp.float32),
            pltpu.VMEM((CH,), jnp.int32),
            pltpu.VMEM((CH,), jnp.int32),
            pltpu.VMEM((CH,), jnp.int32),
            pltpu.VMEM((CH,), jnp.int32),
            pltpu.VMEM((4096,), jnp.int32),
            pltpu.VMEM((256,), jnp.int32),
            pltpu.VMEM((NT, 256), jnp.int32),
            pltpu.VMEM((4096,), jnp.int32),
            pltpu.VMEM((4096,), jnp.int32),
            pltpu.VMEM((NSUB, 128), jnp.int32),
            pltpu.VMEM((CH,), jnp.float32),
            pltpu.VMEM_SHARED((NP,), jnp.int32),
            pltpu.VMEM_SHARED((NP,), jnp.int32),
            pltpu.VMEM_SHARED((NP,), jnp.int32),
            pltpu.VMEM_SHARED((NP,), jnp.int32),
            pltpu.VMEM_SHARED((NT, 256), jnp.int32),
        ],
    )
    def k(rs_hbm, obj_hbm, p0_hbm, p1_hbm,
          t_out, ord_out,
          obj_v, rs_v, p0_v, p1_v, key_v, val_v,
          hist_v, tt_v, grid_v, scanrow_v, offs_v,
          pos2_v, f_v,
          ka_sh, kb_sh, va_sh, vb_sh, hist_sh):
        t = lax.axis_index("s")
        base = t * CH
        lane = lax.iota(jnp.int32, 16)
        ones16 = jnp.ones((16,), jnp.int32)
        zeros16 = jnp.zeros((16,), jnp.int32)

        pltpu.sync_copy(obj_hbm, obj_v)

        @pl.when(t < NT - 1)
        def _full_load():
            pltpu.sync_copy(rs_hbm.at[pl.ds(base, CH)], rs_v)
            pltpu.sync_copy(p0_hbm.at[pl.ds(base, CH)], p0_v)
            pltpu.sync_copy(p1_hbm.at[pl.ds(base, CH)], p1_v)

        @pl.when(t == NT - 1)
        def _tail_load():
            pltpu.sync_copy(rs_hbm.at[pl.ds(base, VALID_LAST)],
                            rs_v.at[pl.ds(0, VALID_LAST)])
            pltpu.sync_copy(p0_hbm.at[pl.ds(base, VALID_LAST)],
                            p0_v.at[pl.ds(0, VALID_LAST)])
            pltpu.sync_copy(p1_hbm.at[pl.ds(base, VALID_LAST)],
                            p1_v.at[pl.ds(0, VALID_LAST)])

        def ph0(j, carry):
            epos = j * 16 + lane
            gidx = base + epos
            pad = gidx >= NUM_REL
            i0 = jnp.where(pad, zeros16, plsc.load_gather(p0_v, [epos]))
            i1 = jnp.where(pad, zeros16, plsc.load_gather(p1_v, [epos]))
            s0 = plsc.load_gather(obj_v, [i0])
            s1 = plsc.load_gather(obj_v, [i1])
            rs = plsc.load_gather(rs_v, [epos])
            tr = (rs * s0) * s1
            bits = plsc.bitcast(tr, jnp.int32)
            key = jnp.where(pad, jnp.int32(0x7FFFFFFF),
                            jnp.int32(0x7FFFFFFF) - bits)
            plsc.store_scatter(key_v, [epos], key)
            plsc.store_scatter(val_v, [epos], jnp.where(pad, zeros16, gidx))
            return carry
        lax.fori_loop(0, LPT, ph0, jnp.int32(0))

        bufs = [(ka_sh, va_sh), (kb_sh, vb_sh)]
        for p in range(4):
            dst = bufs[p % 2]
            if p > 0:
                src = bufs[(p - 1) % 2]
                pltpu.sync_copy(src[0].at[pl.ds(base, CH)], key_v)
                pltpu.sync_copy(src[1].at[pl.ds(base, CH)], val_v)
            shift = 8 * p

            def clr(i, carry):
                plsc.store_scatter(hist_v, [i * 16 + lane], zeros16)
                return carry
            lax.fori_loop(0, 256, clr, jnp.int32(0))

            def hst(j, carry, shift=shift):
                epos = lane * LPT + j
                kk = plsc.load_gather(key_v, [epos])
                d = jnp.bitwise_and(lax.shift_right_logical(kk, shift), 255)
                plsc.addupdate_scatter(hist_v, [d * 16 + lane], ones16)
                return carry
            lax.fori_loop(0, LPT, hst, jnp.int32(0))

            def tot(g, carry):
                dvec = g * 16 + lane
                acc = zeros16
                for l in range(16):
                    acc = acc + plsc.load_gather(hist_v, [dvec * 16 + l])
                plsc.store_scatter(tt_v, [dvec], acc)
                return carry
            lax.fori_loop(0, 16, tot, jnp.int32(0))

            pltpu.sync_copy(tt_v, hist_sh.at[t])
            plsc.subcore_barrier()
            pltpu.sync_copy(hist_sh, grid_v)

            def scn(d, running):
                v = plsc.load_gather(grid_v, [lane, zeros16 + d])
                cs = plsc.cumsum(v)
                basev = (cs - v) + running
                plsc.store_scatter(scanrow_v, [d * 16 + lane], basev)
                lane_h = plsc.load_gather(hist_v, [d * 16 + lane])
                lane_cs = plsc.cumsum(lane_h)
                own = plsc.load_gather(scanrow_v, [zeros16 + (d * 16 + t)])
                plsc.store_scatter(offs_v, [d * 16 + lane],
                                   own + (lane_cs - lane_h))
                return running + jnp.sum(v)
            lax.fori_loop(0, 256, scn, jnp.int32(0))

            def prm(j, carry, shift=shift):
                epos = lane * LPT + j
                kk = plsc.load_gather(key_v, [epos])
                d = jnp.bitwise_and(lax.shift_right_logical(kk, shift), 255)
                oidx = d * 16 + lane
                pos = plsc.load_gather(offs_v, [oidx])
                plsc.store_scatter(offs_v, [oidx], pos + 1)
                plsc.store_scatter(
                    pos2_v,
                    [lax.shift_right_logical(epos, 7),
                     jnp.bitwise_and(epos, 127)], pos)
                return carry
            lax.fori_loop(0, LPT, prm, jnp.int32(0))

            for i in range(NSUB):
                pltpu.sync_copy(key_v.at[pl.ds(i * 128, 128)],
                                dst[0].at[pos2_v.at[i]])
                pltpu.sync_copy(val_v.at[pl.ds(i * 128, 128)],
                                dst[1].at[pos2_v.at[i]])
            plsc.subcore_barrier()

        fk, fv = bufs[3 % 2]
        pltpu.sync_copy(fk.at[pl.ds(base, CH)], key_v)

        def fin(j, carry):
            epos = j * 16 + lane
            kk = plsc.load_gather(key_v, [epos])
            tr = plsc.bitcast(jnp.int32(0x7FFFFFFF) - kk, jnp.float32)
            plsc.store_scatter(f_v, [epos], tr)
            return carry
        lax.fori_loop(0, LPT, fin, jnp.int32(0))
        pltpu.sync_copy(f_v, t_out.at[pl.ds(base, CH)])
        pltpu.sync_copy(fv.at[pl.ds(base, CH)], ord_out.at[pl.ds(base, CH)])

    return k(rel_scores, obj_scores, p0, p1)


def _sc_gather(order, p0, p1, rel_class, prob):
    mesh = plsc.VectorSubcoreMesh(
        core_axis_name="c", subcore_axis_name="s", num_cores=1)

    @functools.partial(
        pl.kernel,
        out_type=(
            jax.ShapeDtypeStruct((NP,), jnp.int32),
            jax.ShapeDtypeStruct((NP,), jnp.int32),
            jax.ShapeDtypeStruct((NP,), jnp.int32),
            jax.ShapeDtypeStruct((NP, PROBW), jnp.float32),
        ),
        mesh=mesh,
        compiler_params=_SC_PARAMS,
        scratch_types=[
            pltpu.VMEM((CH,), jnp.int32),
            pltpu.VMEM((NSUB, 128), jnp.int32),
            pltpu.VMEM((CH,), jnp.int32),
            pltpu.VMEM((CH, PROBW), jnp.float32),
        ],
    )
    def k(ord_hbm, p0_hbm, p1_hbm, cls_hbm, prob_hbm,
          p0_out, p1_out, cls_out, prob_out,
          ord_v, idx2_v, g_v, rows_v):
        t = lax.axis_index("s")
        base = t * CH
        lane = lax.iota(jnp.int32, 16)

        pltpu.sync_copy(ord_hbm.at[pl.ds(base, CH)], ord_v)

        def mkidx(j, carry):
            epos = j * 16 + lane
            vv = plsc.load_gather(ord_v, [epos])
            plsc.store_scatter(
                idx2_v,
                [lax.shift_right_logical(epos, 7),
                 jnp.bitwise_and(epos, 127)], vv)
            return carry
        lax.fori_loop(0, LPT, mkidx, jnp.int32(0))

        for src_hbm, out_hbm in ((p0_hbm, p0_out), (p1_hbm, p1_out),
                                 (cls_hbm, cls_out)):
            for i in range(NSUB):
                pltpu.sync_copy(src_hbm.at[idx2_v.at[i]],
                                g_v.at[pl.ds(i * 128, 128)])
            pltpu.sync_copy(g_v, out_hbm.at[pl.ds(base, CH)])
        for i in range(NSUB):
            pltpu.sync_copy(prob_hbm.at[idx2_v.at[i]],
                            rows_v.at[pl.ds(i * 128, 128)])
        pltpu.sync_copy(rows_v, prob_out.at[pl.ds(base, CH)])

    return k(order, p0, p1, rel_class, prob)


def kernel(rel_logits, obj_logits, rel_pair_idxs):
    rel_scores, rel_class, obj_scores, obj_pred = _scores_stage(
        rel_logits, obj_logits)
    p0 = rel_pair_idxs[:, 0]
    p1 = rel_pair_idxs[:, 1]
    ts, order = _sc_sort(rel_scores, obj_scores, p0, p1)
    rel_class_prob = _prob_stage(rel_logits)
    p0s, p1s, clss, probs = _sc_gather(order, p0, p1, rel_class,
                                       rel_class_prob)
    pair_sorted = jnp.stack([p0s[:NUM_REL], p1s[:NUM_REL]], axis=1)
    return (obj_pred, obj_scores, pair_sorted,
            probs[:NUM_REL, :NUM_REL_CLS], clss[:NUM_REL], ts[:NUM_REL])


# speedup vs baseline: 1.3583x; 1.1393x over previous
import functools

import jax
import jax.numpy as jnp
from jax import lax
from jax.experimental import pallas as pl
from jax.experimental.pallas import tpu as pltpu
from jax.experimental.pallas import tpu_sc as plsc

NUM_REL = 20000
NUM_OBJ = 1000
NUM_REL_CLS = 51
NUM_OBJ_CLS = 151

NP = 20480
NT = 16
CH = NP // NT
LPT = CH // 16
NSUB = CH // 128
PROBW = 64
VALID_LAST = NUM_REL - (NT - 1) * CH

_SC_PARAMS = pltpu.CompilerParams(
    needs_layout_passes=False, use_tc_tiling_on_sc=False)


def _rowsum(e):
    rows, c = e.shape
    p = ((c + 7) // 8) * 8
    if p != c:
        e = jnp.concatenate([e, jnp.zeros((rows, p - c), e.dtype)], axis=1)
    acc = e[:, 0:8]
    for k in range(1, p // 8):
        acc = acc + e[:, 8 * k:8 * k + 8]
    s4 = acc[:, 0:4] + acc[:, 4:8]
    s2 = s4[:, 0:2] + s4[:, 2:4]
    return s2[:, 0:1] + s2[:, 1:2]


def _scores_body(x_ref, scores_ref, cls_ref):
    x = x_ref[...]
    m = jnp.max(x, axis=-1, keepdims=True)
    e = jnp.exp(x - m)
    p = e / _rowsum(e)
    cols = jax.lax.broadcasted_iota(jnp.int32, p.shape, 1)
    pm = jnp.where(cols >= 1, p, -1.0)
    sc = jnp.max(pm, axis=-1)
    scores_ref[...] = sc[:, None]
    cls_ref[...] = jnp.min(
        jnp.where(pm == sc[:, None], cols, x.shape[1]), axis=-1)[:, None]


def _prob_body(x_ref, prob_ref):
    x = x_ref[...]
    m = jnp.max(x, axis=-1, keepdims=True)
    e = jnp.exp(x - m)
    p = e / _rowsum(e)
    prob_ref[...] = jnp.concatenate(
        [p, jnp.zeros((x.shape[0], PROBW - x.shape[1]), p.dtype)], axis=1)


_REL_BLK = 2000


def _scores_stage(rel_logits, obj_logits):
    rel_scores, rel_class = pl.pallas_call(
        _scores_body,
        grid=(NUM_REL // _REL_BLK,),
        in_specs=[pl.BlockSpec((_REL_BLK, NUM_REL_CLS), lambda i: (i, 0))],
        out_specs=(
            pl.BlockSpec((_REL_BLK, 1), lambda i: (i, 0)),
            pl.BlockSpec((_REL_BLK, 1), lambda i: (i, 0)),
        ),
        out_shape=(
            jax.ShapeDtypeStruct((NUM_REL, 1), jnp.float32),
            jax.ShapeDtypeStruct((NUM_REL, 1), jnp.int32),
        ),
    )(rel_logits)
    obj_scores, obj_pred = pl.pallas_call(
        _scores_body,
        out_shape=(
            jax.ShapeDtypeStruct((NUM_OBJ, 1), jnp.float32),
            jax.ShapeDtypeStruct((NUM_OBJ, 1), jnp.int32),
        ),
    )(obj_logits)
    return (rel_scores[:, 0], rel_class[:, 0], obj_scores[:, 0],
            obj_pred[:, 0])


def _prob_stage(rel_logits):
    return pl.pallas_call(
        _prob_body,
        grid=(NUM_REL // _REL_BLK,),
        in_specs=[pl.BlockSpec((_REL_BLK, NUM_REL_CLS), lambda i: (i, 0))],
        out_specs=pl.BlockSpec((_REL_BLK, PROBW), lambda i: (i, 0)),
        out_shape=jax.ShapeDtypeStruct((NUM_REL, PROBW), jnp.float32),
    )(rel_logits)


def _sc_sort(rel_scores, obj_scores, p0, p1):
    mesh = plsc.VectorSubcoreMesh(
        core_axis_name="c", subcore_axis_name="s", num_cores=1)

    @functools.partial(
        pl.kernel,
        out_type=(
            jax.ShapeDtypeStruct((NP,), jnp.float32),
            jax.ShapeDtypeStruct((NP,), jnp.int32),
        ),
        mesh=mesh,
        compiler_params=_SC_PARAMS,
        scratch_types=[
            pltpu.VMEM((NUM_OBJ,), jnp.float32),
            pltpu.VMEM((CH,), jnp.float32),
            pltpu.VMEM((CH,), jnp.int32),
            pltpu.VMEM((CH,), jnp.int32),
            pltpu.VMEM((CH,), jnp.int32),
            pltpu.VMEM((CH,), jnp.int32),
            pltpu.VMEM((4096,), jnp.int32),
            pltpu.VMEM((256,), jnp.int32),
            pltpu.VMEM((NT, 256), jnp.int32),
            pltpu.VMEM((4096,), jnp.int32),
            pltpu.VMEM((4096,), jnp.int32),
            pltpu.VMEM((NSUB, 128), jnp.int32),
            pltpu.VMEM((CH,), jnp.float32),
            pltpu.SemaphoreType.DMA,
            pltpu.VMEM_SHARED((NP,), jnp.int32),
            pltpu.VMEM_SHARED((NP,), jnp.int32),
            pltpu.VMEM_SHARED((NP,), jnp.int32),
            pltpu.VMEM_SHARED((NP,), jnp.int32),
            pltpu.VMEM_SHARED((NT, 256), jnp.int32),
        ],
    )
    def k(rs_hbm, obj_hbm, p0_hbm, p1_hbm,
          t_out, ord_out,
          obj_v, rs_v, p0_v, p1_v, key_v, val_v,
          hist_v, tt_v, grid_v, scanrow_v, offs_v,
          pos2_v, f_v, sem,
          ka_sh, kb_sh, va_sh, vb_sh, hist_sh):
        t = lax.axis_index("s")
        base = t * CH
        lane = lax.iota(jnp.int32, 16)
        ones16 = jnp.ones((16,), jnp.int32)
        zeros16 = jnp.zeros((16,), jnp.int32)

        pltpu.sync_copy(obj_hbm, obj_v)

        @pl.when(t < NT - 1)
        def _full_load():
            pltpu.sync_copy(rs_hbm.at[pl.ds(base, CH)], rs_v)
            pltpu.sync_copy(p0_hbm.at[pl.ds(base, CH)], p0_v)
            pltpu.sync_copy(p1_hbm.at[pl.ds(base, CH)], p1_v)

        @pl.when(t == NT - 1)
        def _tail_load():
            pltpu.sync_copy(rs_hbm.at[pl.ds(base, VALID_LAST)],
                            rs_v.at[pl.ds(0, VALID_LAST)])
            pltpu.sync_copy(p0_hbm.at[pl.ds(base, VALID_LAST)],
                            p0_v.at[pl.ds(0, VALID_LAST)])
            pltpu.sync_copy(p1_hbm.at[pl.ds(base, VALID_LAST)],
                            p1_v.at[pl.ds(0, VALID_LAST)])

        def ph0(j, carry):
            epos = j * 16 + lane
            gidx = base + epos
            pad = gidx >= NUM_REL
            i0 = jnp.where(pad, zeros16, plsc.load_gather(p0_v, [epos]))
            i1 = jnp.where(pad, zeros16, plsc.load_gather(p1_v, [epos]))
            s0 = plsc.load_gather(obj_v, [i0])
            s1 = plsc.load_gather(obj_v, [i1])
            rs = plsc.load_gather(rs_v, [epos])
            tr = (rs * s0) * s1
            bits = plsc.bitcast(tr, jnp.int32)
            key = jnp.where(pad, jnp.int32(0x7FFFFFFF),
                            jnp.int32(0x7FFFFFFF) - bits)
            plsc.store_scatter(key_v, [epos], key)
            plsc.store_scatter(val_v, [epos], jnp.where(pad, zeros16, gidx))
            return carry
        lax.fori_loop(0, LPT, ph0, jnp.int32(0))

        bufs = [(ka_sh, va_sh), (kb_sh, vb_sh)]
        for p in range(4):
            dst = bufs[p % 2]
            if p > 0:
                src = bufs[(p - 1) % 2]
                pltpu.sync_copy(src[0].at[pl.ds(base, CH)], key_v)
                pltpu.sync_copy(src[1].at[pl.ds(base, CH)], val_v)
            shift = 8 * p

            def clr(i, carry):
                plsc.store_scatter(hist_v, [i * 16 + lane], zeros16)
                return carry
            lax.fori_loop(0, 256, clr, jnp.int32(0))

            def hst(j, carry, shift=shift):
                epos = lane * LPT + j
                kk = plsc.load_gather(key_v, [epos])
                d = jnp.bitwise_and(lax.shift_right_logical(kk, shift), 255)
                plsc.addupdate_scatter(hist_v, [d * 16 + lane], ones16)
                return carry
            lax.fori_loop(0, LPT, hst, jnp.int32(0))

            def tot(g, carry):
                dvec = g * 16 + lane
                acc = zeros16
                for l in range(16):
                    acc = acc + plsc.load_gather(hist_v, [dvec * 16 + l])
                plsc.store_scatter(tt_v, [dvec], acc)
                return carry
            lax.fori_loop(0, 16, tot, jnp.int32(0))

            pltpu.sync_copy(tt_v, hist_sh.at[t])
            plsc.subcore_barrier()
            pltpu.sync_copy(hist_sh, grid_v)

            def scn(d, running):
                v = plsc.load_gather(grid_v, [lane, zeros16 + d])
                cs = plsc.cumsum(v)
                basev = (cs - v) + running
                plsc.store_scatter(scanrow_v, [d * 16 + lane], basev)
                lane_h = plsc.load_gather(hist_v, [d * 16 + lane])
                lane_cs = plsc.cumsum(lane_h)
                own = plsc.load_gather(scanrow_v, [zeros16 + (d * 16 + t)])
                plsc.store_scatter(offs_v, [d * 16 + lane],
                                   own + (lane_cs - lane_h))
                return running + jnp.sum(v)
            lax.fori_loop(0, 256, scn, jnp.int32(0))

            def prm(j, carry, shift=shift):
                epos = lane * LPT + j
                kk = plsc.load_gather(key_v, [epos])
                d = jnp.bitwise_and(lax.shift_right_logical(kk, shift), 255)
                oidx = d * 16 + lane
                pos = plsc.load_gather(offs_v, [oidx])
                plsc.store_scatter(offs_v, [oidx], pos + 1)
                plsc.store_scatter(
                    pos2_v,
                    [lax.shift_right_logical(epos, 7),
                     jnp.bitwise_and(epos, 127)], pos)
                return carry
            lax.fori_loop(0, LPT, prm, jnp.int32(0))

            hs = []
            for i in range(NSUB):
                hs.append(pltpu.async_copy(key_v.at[pl.ds(i * 128, 128)],
                                           dst[0].at[pos2_v.at[i]], sem))
                hs.append(pltpu.async_copy(val_v.at[pl.ds(i * 128, 128)],
                                           dst[1].at[pos2_v.at[i]], sem))
            for h in hs:
                h.wait()
            plsc.subcore_barrier()

        fk, fv = bufs[3 % 2]
        pltpu.sync_copy(fk.at[pl.ds(base, CH)], key_v)

        def fin(j, carry):
            epos = j * 16 + lane
            kk = plsc.load_gather(key_v, [epos])
            tr = plsc.bitcast(jnp.int32(0x7FFFFFFF) - kk, jnp.float32)
            plsc.store_scatter(f_v, [epos], tr)
            return carry
        lax.fori_loop(0, LPT, fin, jnp.int32(0))
        pltpu.sync_copy(f_v, t_out.at[pl.ds(base, CH)])
        pltpu.sync_copy(fv.at[pl.ds(base, CH)], ord_out.at[pl.ds(base, CH)])

    return k(rel_scores, obj_scores, p0, p1)


def _sc_gather(order, p0, p1, rel_class, prob):
    mesh = plsc.VectorSubcoreMesh(
        core_axis_name="c", subcore_axis_name="s", num_cores=1)

    @functools.partial(
        pl.kernel,
        out_type=(
            jax.ShapeDtypeStruct((NP,), jnp.int32),
            jax.ShapeDtypeStruct((NP,), jnp.int32),
            jax.ShapeDtypeStruct((NP,), jnp.int32),
            jax.ShapeDtypeStruct((NP, PROBW), jnp.float32),
        ),
        mesh=mesh,
        compiler_params=_SC_PARAMS,
        scratch_types=[
            pltpu.VMEM((CH,), jnp.int32),
            pltpu.VMEM((NSUB, 128), jnp.int32),
            pltpu.VMEM((CH,), jnp.int32),
            pltpu.VMEM((CH,), jnp.int32),
            pltpu.VMEM((CH,), jnp.int32),
            pltpu.VMEM((CH, PROBW), jnp.float32),
            pltpu.SemaphoreType.DMA,
        ],
    )
    def k(ord_hbm, p0_hbm, p1_hbm, cls_hbm, prob_hbm,
          p0_out, p1_out, cls_out, prob_out,
          ord_v, idx2_v, g0_v, g1_v, gc_v, rows_v, sem):
        t = lax.axis_index("s")
        base = t * CH
        lane = lax.iota(jnp.int32, 16)

        pltpu.sync_copy(ord_hbm.at[pl.ds(base, CH)], ord_v)

        def mkidx(j, carry):
            epos = j * 16 + lane
            vv = plsc.load_gather(ord_v, [epos])
            plsc.store_scatter(
                idx2_v,
                [lax.shift_right_logical(epos, 7),
                 jnp.bitwise_and(epos, 127)], vv)
            return carry
        lax.fori_loop(0, LPT, mkidx, jnp.int32(0))

        hs = []
        for src_hbm, stage in ((p0_hbm, g0_v), (p1_hbm, g1_v),
                               (cls_hbm, gc_v), (prob_hbm, rows_v)):
            for i in range(NSUB):
                hs.append(pltpu.async_copy(
                    src_hbm.at[idx2_v.at[i]],
                    stage.at[pl.ds(i * 128, 128)], sem))
        for h in hs:
            h.wait()
        hs = []
        for stage, out_hbm, b, n in (
                (g0_v, p0_out, base, CH), (g1_v, p1_out, base, CH),
                (gc_v, cls_out, base, CH), (rows_v, prob_out, base, CH)):
            hs.append(pltpu.async_copy(stage, out_hbm.at[pl.ds(b, n)], sem))
        for h in hs:
            h.wait()

    return k(order, p0, p1, rel_class, prob)


def kernel(rel_logits, obj_logits, rel_pair_idxs):
    rel_scores, rel_class, obj_scores, obj_pred = _scores_stage(
        rel_logits, obj_logits)
    p0 = rel_pair_idxs[:, 0]
    p1 = rel_pair_idxs[:, 1]
    ts, order = _sc_sort(rel_scores, obj_scores, p0, p1)
    rel_class_prob = _prob_stage(rel_logits)
    p0s, p1s, clss, probs = _sc_gather(order, p0, p1, rel_class,
                                       rel_class_prob)
    pair_sorted = jnp.stack([p0s[:NUM_REL], p1s[:NUM_REL]], axis=1)
    return (obj_pred, obj_scores, pair_sorted,
            probs[:NUM_REL, :NUM_REL_CLS], clss[:NUM_REL], ts[:NUM_REL])


# speedup vs baseline: 1.3615x; 1.0024x over previous
import functools

import jax
import jax.numpy as jnp
from jax import lax
from jax.experimental import pallas as pl
from jax.experimental.pallas import tpu as pltpu
from jax.experimental.pallas import tpu_sc as plsc

NUM_REL = 20000
NUM_OBJ = 1000
NUM_REL_CLS = 51
NUM_OBJ_CLS = 151

NP = 20480
NT = 16
CH = NP // NT
LPT = CH // 16
NSUB = CH // 128
PROBW = 64
VALID_LAST = NUM_REL - (NT - 1) * CH

_SC_PARAMS = pltpu.CompilerParams(
    needs_layout_passes=False, use_tc_tiling_on_sc=False)


def _rowsum(e):
    rows, c = e.shape
    p = ((c + 7) // 8) * 8
    if p != c:
        e = jnp.concatenate([e, jnp.zeros((rows, p - c), e.dtype)], axis=1)
    acc = e[:, 0:8]
    for k in range(1, p // 8):
        acc = acc + e[:, 8 * k:8 * k + 8]
    s4 = acc[:, 0:4] + acc[:, 4:8]
    s2 = s4[:, 0:2] + s4[:, 2:4]
    return s2[:, 0:1] + s2[:, 1:2]


def _scores_body(x_ref, scores_ref, cls_ref):
    x = x_ref[...]
    m = jnp.max(x, axis=-1, keepdims=True)
    e = jnp.exp(x - m)
    p = e / _rowsum(e)
    cols = jax.lax.broadcasted_iota(jnp.int32, p.shape, 1)
    pm = jnp.where(cols >= 1, p, -1.0)
    sc = jnp.max(pm, axis=-1)
    scores_ref[...] = sc[:, None]
    cls_ref[...] = jnp.min(
        jnp.where(pm == sc[:, None], cols, x.shape[1]), axis=-1)[:, None]


def _prob_body(x_ref, prob_ref):
    x = x_ref[...]
    m = jnp.max(x, axis=-1, keepdims=True)
    e = jnp.exp(x - m)
    p = e / _rowsum(e)
    prob_ref[...] = jnp.concatenate(
        [p, jnp.zeros((x.shape[0], PROBW - x.shape[1]), p.dtype)], axis=1)


_REL_BLK = 4000


def _scores_stage(rel_logits, obj_logits):
    rel_scores, rel_class = pl.pallas_call(
        _scores_body,
        grid=(NUM_REL // _REL_BLK,),
        in_specs=[pl.BlockSpec((_REL_BLK, NUM_REL_CLS), lambda i: (i, 0))],
        out_specs=(
            pl.BlockSpec((_REL_BLK, 1), lambda i: (i, 0)),
            pl.BlockSpec((_REL_BLK, 1), lambda i: (i, 0)),
        ),
        out_shape=(
            jax.ShapeDtypeStruct((NUM_REL, 1), jnp.float32),
            jax.ShapeDtypeStruct((NUM_REL, 1), jnp.int32),
        ),
    )(rel_logits)
    obj_scores, obj_pred = pl.pallas_call(
        _scores_body,
        out_shape=(
            jax.ShapeDtypeStruct((NUM_OBJ, 1), jnp.float32),
            jax.ShapeDtypeStruct((NUM_OBJ, 1), jnp.int32),
        ),
    )(obj_logits)
    return (rel_scores[:, 0], rel_class[:, 0], obj_scores[:, 0],
            obj_pred[:, 0])


def _prob_stage(rel_logits):
    return pl.pallas_call(
        _prob_body,
        grid=(NUM_REL // _REL_BLK,),
        in_specs=[pl.BlockSpec((_REL_BLK, NUM_REL_CLS), lambda i: (i, 0))],
        out_specs=pl.BlockSpec((_REL_BLK, PROBW), lambda i: (i, 0)),
        out_shape=jax.ShapeDtypeStruct((NUM_REL, PROBW), jnp.float32),
    )(rel_logits)


def _sc_sort(rel_scores, obj_scores, p0, p1):
    mesh = plsc.VectorSubcoreMesh(
        core_axis_name="c", subcore_axis_name="s", num_cores=1)

    @functools.partial(
        pl.kernel,
        out_type=(
            jax.ShapeDtypeStruct((NP,), jnp.float32),
            jax.ShapeDtypeStruct((NP,), jnp.int32),
        ),
        mesh=mesh,
        compiler_params=_SC_PARAMS,
        scratch_types=[
            pltpu.VMEM((NUM_OBJ,), jnp.float32),
            pltpu.VMEM((CH,), jnp.float32),
            pltpu.VMEM((CH,), jnp.int32),
            pltpu.VMEM((CH,), jnp.int32),
            pltpu.VMEM((CH,), jnp.int32),
            pltpu.VMEM((CH,), jnp.int32),
            pltpu.VMEM((4096,), jnp.int32),
            pltpu.VMEM((256,), jnp.int32),
            pltpu.VMEM((NT, 256), jnp.int32),
            pltpu.VMEM((4096,), jnp.int32),
            pltpu.VMEM((4096,), jnp.int32),
            pltpu.VMEM((NSUB, 128), jnp.int32),
            pltpu.VMEM((CH,), jnp.float32),
            pltpu.SemaphoreType.DMA,
            pltpu.VMEM_SHARED((NP,), jnp.int32),
            pltpu.VMEM_SHARED((NP,), jnp.int32),
            pltpu.VMEM_SHARED((NP,), jnp.int32),
            pltpu.VMEM_SHARED((NP,), jnp.int32),
            pltpu.VMEM_SHARED((NT, 256), jnp.int32),
        ],
    )
    def k(rs_hbm, obj_hbm, p0_hbm, p1_hbm,
          t_out, ord_out,
          obj_v, rs_v, p0_v, p1_v, key_v, val_v,
          hist_v, tt_v, grid_v, scanrow_v, offs_v,
          pos2_v, f_v, sem,
          ka_sh, kb_sh, va_sh, vb_sh, hist_sh):
        t = lax.axis_index("s")
        base = t * CH
        lane = lax.iota(jnp.int32, 16)
        ones16 = jnp.ones((16,), jnp.int32)
        zeros16 = jnp.zeros((16,), jnp.int32)

        pltpu.sync_copy(obj_hbm, obj_v)

        @pl.when(t < NT - 1)
        def _full_load():
            pltpu.sync_copy(rs_hbm.at[pl.ds(base, CH)], rs_v)
            pltpu.sync_copy(p0_hbm.at[pl.ds(base, CH)], p0_v)
            pltpu.sync_copy(p1_hbm.at[pl.ds(base, CH)], p1_v)

        @pl.when(t == NT - 1)
        def _tail_load():
            pltpu.sync_copy(rs_hbm.at[pl.ds(base, VALID_LAST)],
                            rs_v.at[pl.ds(0, VALID_LAST)])
            pltpu.sync_copy(p0_hbm.at[pl.ds(base, VALID_LAST)],
                            p0_v.at[pl.ds(0, VALID_LAST)])
            pltpu.sync_copy(p1_hbm.at[pl.ds(base, VALID_LAST)],
                            p1_v.at[pl.ds(0, VALID_LAST)])

        def ph0(j, carry):
            epos = j * 16 + lane
            gidx = base + epos
            pad = gidx >= NUM_REL
            i0 = jnp.where(pad, zeros16, plsc.load_gather(p0_v, [epos]))
            i1 = jnp.where(pad, zeros16, plsc.load_gather(p1_v, [epos]))
            s0 = plsc.load_gather(obj_v, [i0])
            s1 = plsc.load_gather(obj_v, [i1])
            rs = plsc.load_gather(rs_v, [epos])
            tr = (rs * s0) * s1
            bits = plsc.bitcast(tr, jnp.int32)
            key = jnp.where(pad, jnp.int32(0x7FFFFFFF),
                            jnp.int32(0x7FFFFFFF) - bits)
            plsc.store_scatter(key_v, [epos], key)
            plsc.store_scatter(val_v, [epos], jnp.where(pad, zeros16, gidx))
            return carry
        lax.fori_loop(0, LPT, ph0, jnp.int32(0))

        bufs = [(ka_sh, va_sh), (kb_sh, vb_sh)]
        for p in range(4):
            dst = bufs[p % 2]
            if p > 0:
                src = bufs[(p - 1) % 2]
                pltpu.sync_copy(src[0].at[pl.ds(base, CH)], key_v)
                pltpu.sync_copy(src[1].at[pl.ds(base, CH)], val_v)
            shift = 8 * p

            def clr(i, carry):
                plsc.store_scatter(hist_v, [i * 16 + lane], zeros16)
                return carry
            lax.fori_loop(0, 256, clr, jnp.int32(0))

            def hst(j, carry, shift=shift):
                epos = lane * LPT + j
                kk = plsc.load_gather(key_v, [epos])
                d = jnp.bitwise_and(lax.shift_right_logical(kk, shift), 255)
                plsc.addupdate_scatter(hist_v, [d * 16 + lane], ones16)
                return carry
            lax.fori_loop(0, LPT, hst, jnp.int32(0))

            def tot(g, carry):
                dvec = g * 16 + lane
                acc = zeros16
                for l in range(16):
                    acc = acc + plsc.load_gather(hist_v, [dvec * 16 + l])
                plsc.store_scatter(tt_v, [dvec], acc)
                return carry
            lax.fori_loop(0, 16, tot, jnp.int32(0))

            pltpu.sync_copy(tt_v, hist_sh.at[t])
            plsc.subcore_barrier()
            pltpu.sync_copy(hist_sh, grid_v)

            def scn(d, running):
                v = plsc.load_gather(grid_v, [lane, zeros16 + d])
                cs = plsc.cumsum(v)
                basev = (cs - v) + running
                plsc.store_scatter(scanrow_v, [d * 16 + lane], basev)
                lane_h = plsc.load_gather(hist_v, [d * 16 + lane])
                lane_cs = plsc.cumsum(lane_h)
                own = plsc.load_gather(scanrow_v, [zeros16 + (d * 16 + t)])
                plsc.store_scatter(offs_v, [d * 16 + lane],
                                   own + (lane_cs - lane_h))
                return running + jnp.sum(v)
            lax.fori_loop(0, 256, scn, jnp.int32(0))

            def prm(j, carry, shift=shift):
                epos = lane * LPT + j
                kk = plsc.load_gather(key_v, [epos])
                d = jnp.bitwise_and(lax.shift_right_logical(kk, shift), 255)
                oidx = d * 16 + lane
                pos = plsc.load_gather(offs_v, [oidx])
                plsc.store_scatter(offs_v, [oidx], pos + 1)
                plsc.store_scatter(
                    pos2_v,
                    [lax.shift_right_logical(epos, 7),
                     jnp.bitwise_and(epos, 127)], pos)
                return carry
            lax.fori_loop(0, LPT, prm, jnp.int32(0))

            hs = []
            for i in range(NSUB):
                hs.append(pltpu.async_copy(key_v.at[pl.ds(i * 128, 128)],
                                           dst[0].at[pos2_v.at[i]], sem))
                hs.append(pltpu.async_copy(val_v.at[pl.ds(i * 128, 128)],
                                           dst[1].at[pos2_v.at[i]], sem))
            for h in hs:
                h.wait()
            plsc.subcore_barrier()

        fk, fv = bufs[3 % 2]
        pltpu.sync_copy(fk.at[pl.ds(base, CH)], key_v)

        def fin(j, carry):
            epos = j * 16 + lane
            kk = plsc.load_gather(key_v, [epos])
            tr = plsc.bitcast(jnp.int32(0x7FFFFFFF) - kk, jnp.float32)
            plsc.store_scatter(f_v, [epos], tr)
            return carry
        lax.fori_loop(0, LPT, fin, jnp.int32(0))
        pltpu.sync_copy(f_v, t_out.at[pl.ds(base, CH)])
        pltpu.sync_copy(fv.at[pl.ds(base, CH)], ord_out.at[pl.ds(base, CH)])

    return k(rel_scores, obj_scores, p0, p1)


def _sc_gather(order, p0, p1, rel_class, prob):
    mesh = plsc.VectorSubcoreMesh(
        core_axis_name="c", subcore_axis_name="s", num_cores=1)

    @functools.partial(
        pl.kernel,
        out_type=(
            jax.ShapeDtypeStruct((NP,), jnp.int32),
            jax.ShapeDtypeStruct((NP,), jnp.int32),
            jax.ShapeDtypeStruct((NP,), jnp.int32),
            jax.ShapeDtypeStruct((NP, PROBW), jnp.float32),
        ),
        mesh=mesh,
        compiler_params=_SC_PARAMS,
        scratch_types=[
            pltpu.VMEM((CH,), jnp.int32),
            pltpu.VMEM((NSUB, 128), jnp.int32),
            pltpu.VMEM((CH,), jnp.int32),
            pltpu.VMEM((CH,), jnp.int32),
            pltpu.VMEM((CH,), jnp.int32),
            pltpu.VMEM((CH, PROBW), jnp.float32),
            pltpu.SemaphoreType.DMA,
        ],
    )
    def k(ord_hbm, p0_hbm, p1_hbm, cls_hbm, prob_hbm,
          p0_out, p1_out, cls_out, prob_out,
          ord_v, idx2_v, g0_v, g1_v, gc_v, rows_v, sem):
        t = lax.axis_index("s")
        base = t * CH
        lane = lax.iota(jnp.int32, 16)

        pltpu.sync_copy(ord_hbm.at[pl.ds(base, CH)], ord_v)

        def mkidx(j, carry):
            epos = j * 16 + lane
            vv = plsc.load_gather(ord_v, [epos])
            plsc.store_scatter(
                idx2_v,
                [lax.shift_right_logical(epos, 7),
                 jnp.bitwise_and(epos, 127)], vv)
            return carry
        lax.fori_loop(0, LPT, mkidx, jnp.int32(0))

        hs = []
        for src_hbm, stage in ((p0_hbm, g0_v), (p1_hbm, g1_v),
                               (cls_hbm, gc_v), (prob_hbm, rows_v)):
            for i in range(NSUB):
                hs.append(pltpu.async_copy(
                    src_hbm.at[idx2_v.at[i]],
                    stage.at[pl.ds(i * 128, 128)], sem))
        for h in hs:
            h.wait()
        hs = []
        for stage, out_hbm, b, n in (
                (g0_v, p0_out, base, CH), (g1_v, p1_out, base, CH),
                (gc_v, cls_out, base, CH), (rows_v, prob_out, base, CH)):
            hs.append(pltpu.async_copy(stage, out_hbm.at[pl.ds(b, n)], sem))
        for h in hs:
            h.wait()

    return k(order, p0, p1, rel_class, prob)


def kernel(rel_logits, obj_logits, rel_pair_idxs):
    rel_scores, rel_class, obj_scores, obj_pred = _scores_stage(
        rel_logits, obj_logits)
    p0 = rel_pair_idxs[:, 0]
    p1 = rel_pair_idxs[:, 1]
    ts, order = _sc_sort(rel_scores, obj_scores, p0, p1)
    rel_class_prob = _prob_stage(rel_logits)
    p0s, p1s, clss, probs = _sc_gather(order, p0, p1, rel_class,
                                       rel_class_prob)
    pair_sorted = jnp.stack([p0s[:NUM_REL], p1s[:NUM_REL]], axis=1)
    return (obj_pred, obj_scores, pair_sorted,
            probs[:NUM_REL, :NUM_REL_CLS], clss[:NUM_REL], ts[:NUM_REL])
